# Initial kernel scaffold; baseline (speedup 1.0000x reference)
import jax
import jax.numpy as jnp
from jax.experimental import pallas as pl


def kernel(x, weights, edge_index, W_gcn, b_gcn, W_sage_self, W_sage_neigh, b_sage, W_smax_self, W_smax_neigh, b_smax, W_gin, b_gin, W_lin, b_lin):
    raise NotImplementedError("write your pallas kernel here")


# speedup vs baseline: 1.4452x; 1.4452x over previous
import functools

import jax
import jax.numpy as jnp
from jax.experimental import pallas as pl
from jax.experimental.pallas import tpu as pltpu

N = 10000
E = 320000
D = 128
C = 512
NCHUNK = E // C
T = 1000
NEG = -1e30


def _scatter_kernel(eb_ref, x_ref, cnt_ref, ssum_ref, wsum_ref, dacc_ref,
                    macc_ref, z_ref, dvz_ref):
    p = pl.program_id(0)
    c = pl.program_id(1)

    @pl.when((p == 0) & (c == 0))
    def _init0():
        cnt_ref[...] = jnp.zeros_like(cnt_ref)

    @pl.when(p == 0)
    def _count():
        def body(i, _):
            d = eb_ref[0, 1, i]
            cnt_ref[pl.ds(d, 1), :] += 1.0
            return 0
        jax.lax.fori_loop(0, C, body, 0)

    @pl.when((p == 1) & (c == 0))
    def _init1():
        ssum_ref[...] = jnp.zeros_like(ssum_ref)
        wsum_ref[...] = jnp.zeros_like(wsum_ref)
        dacc_ref[...] = jnp.zeros_like(dacc_ref)
        macc_ref[...] = jnp.full_like(macc_ref, NEG)
        dv = jax.lax.rsqrt(cnt_ref[:, 0:1] + 1.0)
        z_ref[...] = x_ref[...] * dv
        dvz_ref[...] = jnp.broadcast_to(dv, dvz_ref.shape)

    @pl.when(p == 1)
    def _accum():
        def body(i, _):
            s = eb_ref[0, 0, i]
            d = eb_ref[0, 1, i]
            xr = x_ref[pl.ds(s, 1), :]
            zr = z_ref[pl.ds(s, 1), :]
            dr = dvz_ref[pl.ds(s, 1), :]
            ssum_ref[pl.ds(d, 1), :] += xr
            wsum_ref[pl.ds(d, 1), :] += zr
            dacc_ref[pl.ds(d, 1), :] += dr
            macc_ref[pl.ds(d, 1), :] = jnp.maximum(macc_ref[pl.ds(d, 1), :], xr)
            return 0
        jax.lax.fori_loop(0, C, body, 0)


def _dense_kernel(wmix_ref, x_ref, cnt_ref, ssum_ref, wsum_ref, dacc_ref,
                  macc_ref, Wg_ref, bg_ref, Wss_ref, Wsn_ref, bs_ref,
                  Wms_ref, Wmn_ref, bm_ref, Wgin_ref, bgin_ref, Wl_ref,
                  bl_ref, out_ref):
    x = x_ref[...]
    cnt = cnt_ref[:, 0:1]
    dsum = dacc_ref[:, 0:1]
    dinv = jax.lax.rsqrt(cnt + 1.0)
    ssum = ssum_ref[...]

    def mm(a, w_ref):
        return jax.lax.dot_general(
            a, w_ref[...], (((1,), (0,)), ((), ())),
            preferred_element_type=jnp.float32)

    h_self = mm(x, Wg_ref) + bg_ref[...]
    out_gcn = dinv * (mm(wsum_ref[...], Wg_ref) + dsum * bg_ref[...]) \
        + (dinv * dinv) * h_self
    mean = ssum / jnp.maximum(cnt, 1.0)
    out_sage = mm(x, Wss_ref) + mm(mean, Wsn_ref) + bs_ref[...]
    mx = jnp.where(cnt > 0.0, macc_ref[...], 0.0)
    out_smax = mm(x, Wms_ref) + mm(mx, Wmn_ref) + bm_ref[...]
    out_gin = mm(x + ssum, Wgin_ref) + bgin_ref[...]
    out_lin = mm(x, Wl_ref) + bl_ref[...]

    def elu(v):
        return jnp.where(v > 0.0, v, jnp.exp(jnp.minimum(v, 0.0)) - 1.0)

    out_ref[...] = (wmix_ref[0] * elu(out_gcn) + wmix_ref[1] * elu(out_sage)
                    + wmix_ref[2] * elu(out_smax) + wmix_ref[3] * elu(out_gin)
                    + wmix_ref[4] * elu(out_lin))


@functools.partial(jax.jit)
def kernel(x, weights, edge_index, W_gcn, b_gcn, W_sage_self, W_sage_neigh,
           b_sage, W_smax_self, W_smax_neigh, b_smax, W_gin, b_gin, W_lin,
           b_lin):
    eb = edge_index.reshape(2, NCHUNK, C).transpose(1, 0, 2)

    cnt, ssum, wsum, dacc, macc = pl.pallas_call(
        _scatter_kernel,
        grid=(2, NCHUNK),
        in_specs=[
            pl.BlockSpec((1, 2, C), lambda p, c: (c, 0, 0),
                         memory_space=pltpu.SMEM),
            pl.BlockSpec((N, D), lambda p, c: (0, 0)),
        ],
        out_specs=[
            pl.BlockSpec((N, 8), lambda p, c: (0, 0)),
            pl.BlockSpec((N, D), lambda p, c: (0, 0)),
            pl.BlockSpec((N, D), lambda p, c: (0, 0)),
            pl.BlockSpec((N, 8), lambda p, c: (0, 0)),
            pl.BlockSpec((N, D), lambda p, c: (0, 0)),
        ],
        out_shape=[
            jax.ShapeDtypeStruct((N, 8), jnp.float32),
            jax.ShapeDtypeStruct((N, D), jnp.float32),
            jax.ShapeDtypeStruct((N, D), jnp.float32),
            jax.ShapeDtypeStruct((N, 8), jnp.float32),
            jax.ShapeDtypeStruct((N, D), jnp.float32),
        ],
        scratch_shapes=[
            pltpu.VMEM((N, D), jnp.float32),
            pltpu.VMEM((N, 8), jnp.float32),
        ],
        compiler_params=pltpu.CompilerParams(
            dimension_semantics=("arbitrary", "arbitrary")),
    )(eb, x)

    wmix = jnp.pad(weights, (0, 3))
    full = lambda i: (0, 0)
    row = pl.BlockSpec((T, D), lambda i: (i, 0))
    small = pl.BlockSpec((T, 8), lambda i: (i, 0))
    wspec = pl.BlockSpec((D, D), full)
    bspec = pl.BlockSpec((1, D), full)

    out = pl.pallas_call(
        _dense_kernel,
        grid=(N // T,),
        in_specs=[
            pl.BlockSpec(memory_space=pltpu.SMEM),
            row, small, row, row, small, row,
            wspec, bspec, wspec, wspec, bspec,
            wspec, wspec, bspec, wspec, bspec, wspec, bspec,
        ],
        out_specs=row,
        out_shape=jax.ShapeDtypeStruct((N, D), jnp.float32),
        compiler_params=pltpu.CompilerParams(
            dimension_semantics=("arbitrary",)),
    )(wmix, x, cnt, ssum, wsum, dacc, macc,
      W_gcn, b_gcn.reshape(1, D),
      W_sage_self, W_sage_neigh, b_sage.reshape(1, D),
      W_smax_self, W_smax_neigh, b_smax.reshape(1, D),
      W_gin, b_gin.reshape(1, D),
      W_lin, b_lin.reshape(1, D))
    return out


# speedup vs baseline: 2.0264x; 1.4021x over previous
import functools

import jax
import jax.numpy as jnp
from jax.experimental import pallas as pl
from jax.experimental.pallas import tpu as pltpu

N = 10000
E = 320000
D = 128
C = 512
NCHUNK = E // C
T = 1000
NEG = -1e30


H = C // 2


def _scatter_kernel(eb_ref, x_ref, cnt_ref, ssum_ref, wsum_ref, dacc_ref,
                    macc_ref, z_ref, dvz_ref, sacc2, wacc2, macc2):
    p = pl.program_id(0)
    c = pl.program_id(1)

    @pl.when((p == 0) & (c == 0))
    def _init0():
        cnt_ref[...] = jnp.zeros_like(cnt_ref)

    @pl.when(p == 0)
    def _count():
        def body(i, _):
            dA = eb_ref[0, 1, i]
            dB = eb_ref[0, 1, i + H]
            cnt_ref[pl.ds(dA, 1), :] += 1.0
            cnt_ref[pl.ds(dB, 1), :] += 1.0
            return 0
        jax.lax.fori_loop(0, H, body, 0)

    @pl.when((p == 1) & (c == 0))
    def _init1():
        ssum_ref[...] = jnp.zeros_like(ssum_ref)
        wsum_ref[...] = jnp.zeros_like(wsum_ref)
        dacc_ref[...] = jnp.zeros_like(dacc_ref)
        macc_ref[...] = jnp.full_like(macc_ref, NEG)
        sacc2[...] = jnp.zeros_like(sacc2)
        wacc2[...] = jnp.zeros_like(wacc2)
        macc2[...] = jnp.full_like(macc2, NEG)
        dv = jax.lax.rsqrt(cnt_ref[:, 0:1] + 1.0)
        z_ref[...] = x_ref[...] * dv
        dvz_ref[...] = jnp.broadcast_to(dv, dvz_ref.shape)

    @pl.when(p == 1)
    def _accum():
        def body(i, _):
            sA = eb_ref[0, 0, i]
            dA = eb_ref[0, 1, i]
            sB = eb_ref[0, 0, i + H]
            dB = eb_ref[0, 1, i + H]
            xrA = x_ref[pl.ds(sA, 1), :]
            zrA = z_ref[pl.ds(sA, 1), :]
            drA = dvz_ref[pl.ds(sA, 1), :]
            xrB = x_ref[pl.ds(sB, 1), :]
            zrB = z_ref[pl.ds(sB, 1), :]
            drB = dvz_ref[pl.ds(sB, 1), :]
            ssum_ref[pl.ds(dA, 1), :] += xrA
            sacc2[pl.ds(dB, 1), :] += xrB
            wsum_ref[pl.ds(dA, 1), :] += zrA
            wacc2[pl.ds(dB, 1), :] += zrB
            dacc_ref[pl.ds(dA, 1), :] += drA
            dacc_ref[pl.ds(dB, 1), :] += drB
            macc_ref[pl.ds(dA, 1), :] = jnp.maximum(
                macc_ref[pl.ds(dA, 1), :], xrA)
            macc2[pl.ds(dB, 1), :] = jnp.maximum(macc2[pl.ds(dB, 1), :], xrB)
            return 0
        jax.lax.fori_loop(0, H, body, 0)

    @pl.when((p == 1) & (c == NCHUNK - 1))
    def _fold1():
        ssum_ref[...] += sacc2[...]
        wsum_ref[...] += wacc2[...]
        macc_ref[...] = jnp.maximum(macc_ref[...], macc2[...])


def _dense_kernel(wmix_ref, x_ref, cnt_ref, ssum_ref, wsum_ref, dacc_ref,
                  macc_ref, Wg_ref, bg_ref, Wss_ref, Wsn_ref, bs_ref,
                  Wms_ref, Wmn_ref, bm_ref, Wgin_ref, bgin_ref, Wl_ref,
                  bl_ref, out_ref):
    x = x_ref[...]
    cnt = cnt_ref[:, 0:1]
    dsum = dacc_ref[:, 0:1]
    dinv = jax.lax.rsqrt(cnt + 1.0)
    ssum = ssum_ref[...]

    def mm(a, w_ref):
        return jax.lax.dot_general(
            a, w_ref[...], (((1,), (0,)), ((), ())),
            preferred_element_type=jnp.float32)

    h_self = mm(x, Wg_ref) + bg_ref[...]
    out_gcn = dinv * (mm(wsum_ref[...], Wg_ref) + dsum * bg_ref[...]) \
        + (dinv * dinv) * h_self
    mean = ssum / jnp.maximum(cnt, 1.0)
    out_sage = mm(x, Wss_ref) + mm(mean, Wsn_ref) + bs_ref[...]
    mx = jnp.where(cnt > 0.0, macc_ref[...], 0.0)
    out_smax = mm(x, Wms_ref) + mm(mx, Wmn_ref) + bm_ref[...]
    out_gin = mm(x + ssum, Wgin_ref) + bgin_ref[...]
    out_lin = mm(x, Wl_ref) + bl_ref[...]

    def elu(v):
        return jnp.where(v > 0.0, v, jnp.exp(jnp.minimum(v, 0.0)) - 1.0)

    out_ref[...] = (wmix_ref[0] * elu(out_gcn) + wmix_ref[1] * elu(out_sage)
                    + wmix_ref[2] * elu(out_smax) + wmix_ref[3] * elu(out_gin)
                    + wmix_ref[4] * elu(out_lin))


@functools.partial(jax.jit)
def kernel(x, weights, edge_index, W_gcn, b_gcn, W_sage_self, W_sage_neigh,
           b_sage, W_smax_self, W_smax_neigh, b_smax, W_gin, b_gin, W_lin,
           b_lin):
    eb = edge_index.reshape(2, NCHUNK, C).transpose(1, 0, 2)

    cnt, ssum, wsum, dacc, macc = pl.pallas_call(
        _scatter_kernel,
        grid=(2, NCHUNK),
        in_specs=[
            pl.BlockSpec((1, 2, C), lambda p, c: (c, 0, 0),
                         memory_space=pltpu.SMEM),
            pl.BlockSpec((N, D), lambda p, c: (0, 0)),
        ],
        out_specs=[
            pl.BlockSpec((N, 8), lambda p, c: (0, 0)),
            pl.BlockSpec((N, D), lambda p, c: (0, 0)),
            pl.BlockSpec((N, D), lambda p, c: (0, 0)),
            pl.BlockSpec((N, 8), lambda p, c: (0, 0)),
            pl.BlockSpec((N, D), lambda p, c: (0, 0)),
        ],
        out_shape=[
            jax.ShapeDtypeStruct((N, 8), jnp.float32),
            jax.ShapeDtypeStruct((N, D), jnp.float32),
            jax.ShapeDtypeStruct((N, D), jnp.float32),
            jax.ShapeDtypeStruct((N, 8), jnp.float32),
            jax.ShapeDtypeStruct((N, D), jnp.float32),
        ],
        scratch_shapes=[
            pltpu.VMEM((N, D), jnp.float32),
            pltpu.VMEM((N, 8), jnp.float32),
            pltpu.VMEM((N, D), jnp.float32),
            pltpu.VMEM((N, D), jnp.float32),
            pltpu.VMEM((N, D), jnp.float32),
        ],
        compiler_params=pltpu.CompilerParams(
            dimension_semantics=("arbitrary", "arbitrary")),
    )(eb, x)

    wmix = jnp.pad(weights, (0, 3))
    full = lambda i: (0, 0)
    row = pl.BlockSpec((T, D), lambda i: (i, 0))
    small = pl.BlockSpec((T, 8), lambda i: (i, 0))
    wspec = pl.BlockSpec((D, D), full)
    bspec = pl.BlockSpec((1, D), full)

    out = pl.pallas_call(
        _dense_kernel,
        grid=(N // T,),
        in_specs=[
            pl.BlockSpec(memory_space=pltpu.SMEM),
            row, small, row, row, small, row,
            wspec, bspec, wspec, wspec, bspec,
            wspec, wspec, bspec, wspec, bspec, wspec, bspec,
        ],
        out_specs=row,
        out_shape=jax.ShapeDtypeStruct((N, D), jnp.float32),
        compiler_params=pltpu.CompilerParams(
            dimension_semantics=("arbitrary",)),
    )(wmix, x, cnt, ssum, wsum, dacc, macc,
      W_gcn, b_gcn.reshape(1, D),
      W_sage_self, W_sage_neigh, b_sage.reshape(1, D),
      W_smax_self, W_smax_neigh, b_smax.reshape(1, D),
      W_gin, b_gin.reshape(1, D),
      W_lin, b_lin.reshape(1, D))
    return out
